# baseline (speedup 1.0000x reference)
import functools

import jax
import jax.numpy as jnp
from jax import lax
from jax.experimental import pallas as pl
from jax.experimental.pallas import tpu as pltpu
from jax.experimental.pallas import tpu_sc as plsc

B, H, D = 4096, 200, 32
VOCAB = 1000
REP = 16
DQ = 4
NQ = D // DQ
SLICE = DQ * VOCAB * REP

NC = 2
NS = 16
NW = NC * NS
TPQ = NW // NQ
U_PER_W = H // TPQ
NBG = B // 16


@functools.partial(
    pl.kernel,
    out_type=jax.ShapeDtypeStruct((H, D, B), jnp.float32),
    mesh=plsc.VectorSubcoreMesh(
        core_axis_name="c", subcore_axis_name="s", num_cores=NC, num_subcores=NS
    ),
    scratch_types=[
        pltpu.VMEM((SLICE,), jnp.float32),
        pltpu.VMEM((B,), jnp.int32),
        pltpu.VMEM((DQ, B), jnp.float32),
        pltpu.VMEM((DQ, B), jnp.float32),
        pltpu.SemaphoreType.DMA,
    ],
    compiler_params=pltpu.CompilerParams(needs_layout_passes=False),
)
def _gather_kernel(rep_hbm, items_hbm, out_hbm, table_v, idx_v,
                   buf_a, buf_b, sem):
    wid = lax.axis_index("s") * NC + lax.axis_index("c")
    q = wid % NQ
    hbase = wid // NQ
    pltpu.sync_copy(rep_hbm.at[pl.ds(q * SLICE, SLICE)], table_v)

    iota0 = lax.iota(jnp.int32, 16)
    consts = [jnp.full((16,), dd * VOCAB * REP, jnp.int32) for dd in range(DQ)]

    def fill(buf):
        def group2(g2, _):
            for gg in range(2):
                j0 = (g2 * 2 + gg) * 16
                idx16 = idx_v[pl.ds(j0, 16)]
                base = idx16 * REP + iota0
                vals = [plsc.load_gather(table_v, [base + consts[dd]])
                        for dd in range(DQ)]
                for dd in range(DQ):
                    buf[dd, pl.ds(j0, 16)] = vals[dd]
            return 0

        lax.fori_loop(0, NBG // 2, group2, 0)

    def do_unit(i, buf, drain):
        h = hbase + TPQ * i
        pltpu.sync_copy(items_hbm.at[pl.ds(h * B, B)], idx_v)
        dst = out_hbm.at[h, pl.ds(q * DQ, DQ)]
        if drain:
            pltpu.make_async_copy(buf, dst, sem).wait()
        fill(buf)
        pltpu.async_copy(buf, dst, sem)

    def pair_body(p, _):
        do_unit(2 * p, buf_a, True)
        do_unit(2 * p + 1, buf_b, True)
        return 0

    do_unit(0, buf_a, False)
    do_unit(1, buf_b, False)
    lax.fori_loop(1, U_PER_W // 2, pair_body, 0)

    dst0 = out_hbm.at[0, pl.ds(0, DQ)]
    pltpu.make_async_copy(buf_a, dst0, sem).wait()
    pltpu.make_async_copy(buf_b, dst0, sem).wait()


def kernel(items, timesteps, item_embedding_table):
    del timesteps
    items_t = items.T.astype(jnp.int32).reshape(-1)
    rep = jnp.broadcast_to(
        item_embedding_table.T.reshape(D, VOCAB, 1), (D, VOCAB, REP)
    ).reshape(-1)
    out_t = _gather_kernel(rep, items_t)
    return jnp.transpose(out_t, (2, 0, 1))


# speedup vs baseline: 1.2761x; 1.2761x over previous
import functools

import jax
import jax.numpy as jnp
from jax import lax
from jax.experimental import pallas as pl
from jax.experimental.pallas import tpu as pltpu
from jax.experimental.pallas import tpu_sc as plsc

B, H, D = 4096, 200, 32
VOCAB = 1000
REP = 8
DO = 8
NO = D // DO
SLICE = DO * VOCAB * REP
CSLICE = DO * VOCAB
CBASE = SLICE - CSLICE
BH = B // 2

NC = 2
NS = 16
NW = NC * NS
TPO = NW // NO
U_PER_W = H // TPO * 2
NBG = BH // 16


@functools.partial(
    pl.kernel,
    out_type=jax.ShapeDtypeStruct((H, D, B), jnp.float32),
    mesh=plsc.VectorSubcoreMesh(
        core_axis_name="c", subcore_axis_name="s", num_cores=NC, num_subcores=NS
    ),
    scratch_types=[
        pltpu.VMEM((SLICE,), jnp.float32),
        pltpu.VMEM((BH,), jnp.int32),
        pltpu.VMEM((DO, BH), jnp.float32),
        pltpu.VMEM((DO, BH), jnp.float32),
        pltpu.SemaphoreType.DMA,
    ],
    compiler_params=pltpu.CompilerParams(needs_layout_passes=False),
)
def _gather_kernel(table_hbm, items_hbm, out_hbm, rep_v, idx_v,
                   buf_a, buf_b, sem):
    wid = lax.axis_index("s") * NC + lax.axis_index("c")
    q = wid % NO
    hbase = wid // NO
    pltpu.sync_copy(table_hbm.at[pl.ds(q * CSLICE, CSLICE)],
                    rep_v.at[pl.ds(CBASE, CSLICE)])

    iota16 = lax.iota(jnp.int32, 16)
    hi8 = jnp.where(iota16 >= 8, 1, 0)
    pair_pats = [hi8 + 2 * k for k in range(8)]

    def splat_pair(vals, k):
        return vals.at[pair_pats[k]].get(mode="promise_in_bounds")

    def expand(dd, _):
        def blk(g, _):
            vals = rep_v[pl.ds(CBASE + dd * VOCAB + g * 16, 16)]
            outs = [splat_pair(vals, k) for k in range(8)]
            for k in range(8):
                rep_v[pl.ds(dd * (VOCAB * REP) + g * 128 + k * 16, 16)] = \
                    outs[k]
            return 0
        lax.fori_loop(0, VOCAB // 16, blk, 0)
        vals = rep_v[pl.ds(CBASE + dd * VOCAB + VOCAB - 16, 16)]
        for k in range(4):
            rep_v[pl.ds(dd * (VOCAB * REP) + (VOCAB // 16) * 128 + k * 16,
                        16)] = splat_pair(vals, k + 4)
        return 0

    lax.fori_loop(0, DO, expand, 0)

    iota0 = lax.iota(jnp.int32, 16)
    lane8 = jnp.bitwise_and(iota0, 7)
    consts = [jnp.full((16,), dd * VOCAB * REP, jnp.int32) + lane8
              for dd in range(DO)]

    def fill(buf):
        def group2(g2, _):
            for gg in range(2):
                j0 = (g2 * 2 + gg) * 16
                idx16 = idx_v[pl.ds(j0, 16)]
                base = idx16 * REP
                vals = [plsc.load_gather(rep_v, [base + consts[dd]])
                        for dd in range(DO)]
                for dd in range(DO):
                    buf[dd, pl.ds(j0, 16)] = vals[dd]
            return 0

        lax.fori_loop(0, NBG // 2, group2, 0)

    def do_unit(i, buf, drain):
        h = hbase + TPO * (i // 2)
        half = i % 2
        pltpu.sync_copy(items_hbm.at[pl.ds(h * B + half * BH, BH)], idx_v)
        dst = out_hbm.at[h, pl.ds(q * DO, DO), pl.ds(half * BH, BH)]
        if drain:
            pltpu.make_async_copy(buf, dst, sem).wait()
        fill(buf)
        pltpu.async_copy(buf, dst, sem)

    def pair_body(p, _):
        do_unit(2 * p, buf_a, True)
        do_unit(2 * p + 1, buf_b, True)
        return 0

    do_unit(0, buf_a, False)
    do_unit(1, buf_b, False)
    lax.fori_loop(1, U_PER_W // 2, pair_body, 0)

    dst0 = out_hbm.at[0, pl.ds(0, DO), pl.ds(0, BH)]
    pltpu.make_async_copy(buf_a, dst0, sem).wait()
    pltpu.make_async_copy(buf_b, dst0, sem).wait()


def kernel(items, timesteps, item_embedding_table):
    del timesteps
    items_t = items.T.astype(jnp.int32).reshape(-1)
    table_t = item_embedding_table.T.reshape(-1)
    out_t = _gather_kernel(table_t, items_t)
    return jnp.transpose(out_t, (2, 0, 1))


# speedup vs baseline: 1.8830x; 1.4756x over previous
import functools

import jax
import jax.numpy as jnp
from jax import lax
from jax.experimental import pallas as pl
from jax.experimental.pallas import tpu as pltpu
from jax.experimental.pallas import tpu_sc as plsc

B, H, D = 4096, 200, 32
VOCAB = 1000
TV = VOCAB * D

NC = 2
NS = 16
NW = NC * NS
N_UNITS = H * (D // 8)
U_PER_W = N_UNITS // NW
NBG = B // 16


@functools.partial(
    pl.kernel,
    out_type=jax.ShapeDtypeStruct((H, D, B), jnp.float32),
    mesh=plsc.VectorSubcoreMesh(
        core_axis_name="c", subcore_axis_name="s", num_cores=NC, num_subcores=NS
    ),
    scratch_types=[
        pltpu.VMEM((TV,), jnp.float32),
        pltpu.VMEM((B,), jnp.int32),
        pltpu.VMEM((8, B), jnp.float32),
        pltpu.VMEM((8, B), jnp.float32),
        pltpu.SemaphoreType.DMA,
    ],
    compiler_params=pltpu.CompilerParams(needs_layout_passes=False),
)
def _gather_kernel(table_hbm, items_hbm, out_hbm, table_v, idx_v,
                   buf_a, buf_b, sem):
    wid = lax.axis_index("s") * NC + lax.axis_index("c")
    u0 = wid * U_PER_W
    pltpu.sync_copy(table_hbm, table_v)

    def fill(buf, d0):
        consts = [jnp.full((16,), (d0 + dd) * VOCAB, jnp.int32)
                  for dd in range(8)]

        def group2(g2, _):
            for gg in range(2):
                j0 = (g2 * 2 + gg) * 16
                idx16 = idx_v[pl.ds(j0, 16)]
                addrs = [idx16 + consts[dd] for dd in range(8)]
                vals = [plsc.load_gather(table_v, [a]) for a in addrs]
                for dd in range(8):
                    buf[dd, pl.ds(j0, 16)] = vals[dd]
            return 0

        lax.fori_loop(0, NBG // 2, group2, 0)

    def do_unit(u, buf, drain):
        h = u // (D // 8)
        d0 = (u % (D // 8)) * 8

        @pl.when(jnp.logical_or(u % (D // 8) == 0, u == u0))
        def _stage():
            pltpu.sync_copy(items_hbm.at[pl.ds(h * B, B)], idx_v)

        dst = out_hbm.at[h, pl.ds(d0, 8)]
        if drain:
            pltpu.make_async_copy(buf, dst, sem).wait()
        fill(buf, d0)
        pltpu.async_copy(buf, dst, sem)

    def pair_body(i, _):
        for off, buf in ((0, buf_a), (1, buf_b)):
            do_unit(u0 + 2 * i + off, buf, True)
        return 0

    do_unit(u0, buf_a, False)
    do_unit(u0 + 1, buf_b, False)
    lax.fori_loop(1, (U_PER_W - 1) // 2, pair_body, 0)
    do_unit(u0 + U_PER_W - 1, buf_a, True)

    dst0 = out_hbm.at[0, pl.ds(0, 8)]
    pltpu.make_async_copy(buf_a, dst0, sem).wait()
    pltpu.make_async_copy(buf_b, dst0, sem).wait()


def kernel(items, timesteps, item_embedding_table):
    del timesteps
    items_t = items.T.astype(jnp.int32).reshape(-1)
    table_t = item_embedding_table.T.reshape(-1)
    out_t = _gather_kernel(table_t, items_t)
    return jnp.transpose(out_t, (2, 0, 1))
